# Initial kernel scaffold; baseline (speedup 1.0000x reference)
import jax
import jax.numpy as jnp
from jax.experimental import pallas as pl


def kernel(inputs, table):
    raise NotImplementedError("write your pallas kernel here")


# speedup vs baseline: 4.7207x; 4.7207x over previous
import functools

import jax
import jax.numpy as jnp
from jax import lax
from jax.experimental import pallas as pl
from jax.experimental.pallas import tpu as pltpu
from jax.experimental.pallas import tpu_sc as plsc

_SCALE = 11.313708498984761
_D = 128
_C = 128


@functools.partial(jax.jit, static_argnums=(0,))
def _embed(n_rows, idx, table):
    info = plsc.get_sparse_core_info()
    num_cores, num_subcores = info.num_cores, info.num_subcores
    nw = num_cores * num_subcores
    b_per_w = n_rows // nw
    g_chunks = b_per_w // _C

    mesh = plsc.VectorSubcoreMesh(core_axis_name="c", subcore_axis_name="s")

    @functools.partial(
        pl.kernel,
        mesh=mesh,
        out_type=jax.ShapeDtypeStruct((n_rows, _D), jnp.float32),
        scratch_types=[
            pltpu.VMEM((g_chunks, _C), jnp.int32),
            pltpu.VMEM((_C, _D), jnp.float32),
            pltpu.SemaphoreType.DMA,
        ],
    )
    def k(idx_hbm, table_hbm, out_hbm, idx_v, rows_v, sem):
        wid = lax.axis_index("s") * num_cores + lax.axis_index("c")
        pltpu.sync_copy(idx_hbm.at[wid], idx_v)

        def chunk(g, carry):
            pltpu.async_copy(table_hbm.at[idx_v.at[g]], rows_v, sem).wait()

            def row(r, c):
                for j in range(_D // 16):
                    sl = pl.ds(j * 16, 16)
                    rows_v[r, sl] = rows_v[r, sl] * _SCALE
                return c

            lax.fori_loop(0, _C, row, 0)
            pltpu.sync_copy(
                rows_v, out_hbm.at[pl.ds(wid * b_per_w + g * _C, _C)]
            )
            return carry

        lax.fori_loop(0, g_chunks, chunk, 0)

    return k(idx, table)


def kernel(inputs, table):
    shape = inputs.shape
    n = inputs.size
    idx = inputs.reshape(32, -1, _C).astype(jnp.int32)
    out = _embed(n, idx, table)
    return out.reshape(*shape, _D)


# speedup vs baseline: 7.4163x; 1.5710x over previous
import functools

import jax
import jax.numpy as jnp
from jax import lax
from jax.experimental import pallas as pl
from jax.experimental.pallas import tpu as pltpu
from jax.experimental.pallas import tpu_sc as plsc

_SCALE = 11.313708498984761
_D = 128
_C = 128


@functools.partial(jax.jit, static_argnums=(0,))
def _embed(n_rows, idx, table):
    info = plsc.get_sparse_core_info()
    num_cores, num_subcores = info.num_cores, info.num_subcores
    nw = num_cores * num_subcores
    b_per_w = n_rows // nw
    g_chunks = b_per_w // _C

    mesh = plsc.VectorSubcoreMesh(core_axis_name="c", subcore_axis_name="s")

    @functools.partial(
        pl.kernel,
        mesh=mesh,
        out_type=jax.ShapeDtypeStruct((n_rows, _D), jnp.float32),
        scratch_types=[
            pltpu.VMEM((g_chunks, _C), jnp.int32),
            pltpu.VMEM((_C, _D), jnp.float32),
            pltpu.VMEM((_C, _D), jnp.float32),
            pltpu.SemaphoreType.DMA,
            pltpu.SemaphoreType.DMA,
            pltpu.SemaphoreType.DMA,
            pltpu.SemaphoreType.DMA,
        ],
    )
    def k(idx_hbm, table_hbm, out_hbm, idx_v, rows0, rows1, g0, g1, o0, o1):
        wid = lax.axis_index("s") * num_cores + lax.axis_index("c")
        base = wid * b_per_w
        pltpu.sync_copy(idx_hbm.at[wid], idx_v)

        bufs = (rows0, rows1)
        gsems = (g0, g1)
        osems = (o0, o1)

        def start_gather(g, b):
            pltpu.async_copy(table_hbm.at[idx_v.at[g]], bufs[b], gsems[b])

        def wait_gather(g, b):
            pltpu.make_async_copy(
                table_hbm.at[idx_v.at[g]], bufs[b], gsems[b]
            ).wait()

        def start_out(g, b):
            pltpu.async_copy(
                bufs[b], out_hbm.at[pl.ds(base + g * _C, _C)], osems[b]
            )

        def wait_out(g, b):
            pltpu.make_async_copy(
                bufs[b], out_hbm.at[pl.ds(base + g * _C, _C)], osems[b]
            ).wait()

        def scale(b):
            buf = bufs[b]

            def row(r, c):
                for j in range(_D // 16):
                    sl = pl.ds(j * 16, 16)
                    buf[r, sl] = buf[r, sl] * _SCALE
                return c

            lax.fori_loop(0, _C, row, 0)

        start_gather(0, 0)

        def step(i, carry):
            g = 2 * i

            @pl.when(i >= 1)
            def _():
                wait_out(g - 1, 1)

            start_gather(g + 1, 1)
            wait_gather(g, 0)
            scale(0)
            start_out(g, 0)

            @pl.when(g + 2 < g_chunks)
            def _():
                wait_out(g, 0)
                start_gather(g + 2, 0)

            wait_gather(g + 1, 1)
            scale(1)
            start_out(g + 1, 1)
            return carry

        lax.fori_loop(0, g_chunks // 2, step, 0)
        wait_out(g_chunks - 2, 0)
        wait_out(g_chunks - 1, 1)

    return k(idx, table)


def kernel(inputs, table):
    shape = inputs.shape
    n = inputs.size
    idx = inputs.reshape(32, -1, _C).astype(jnp.int32)
    out = _embed(n, idx, table)
    return out.reshape(*shape, _D)
